# baseline (speedup 1.0000x reference)
import functools

import jax
import jax.numpy as jnp
from jax import lax
from jax.experimental import pallas as pl
from jax.experimental.pallas import tpu as pltpu
from jax.experimental.pallas import tpu_sc as plsc

C_IN, D_OUT, N_ETAS, H_GRID, W_GRID = 512, 512, 8, 64, 128
N_COLS = H_GRID * W_GRID

_NC, _NS = 2, 16
_NW = _NC * _NS
_SLICE = N_COLS // _NW
_NV = _SLICE // 16

TILE_M = 256
P_PAD = N_COLS + N_ETAS * TILE_M
NT = P_PAD // TILE_M
NT_PAD = 48

CHUNK = 128

_sc_mesh = plsc.VectorSubcoreMesh(core_axis_name="c", subcore_axis_name="s")


def _wid():
    return lax.axis_index("s") * _NC + lax.axis_index("c")


@functools.partial(
    pl.kernel, mesh=_sc_mesh,
    out_type=jax.ShapeDtypeStruct((_NW * 16,), jnp.int32),
    scratch_types=[pltpu.VMEM((_SLICE,), jnp.int32),
                   pltpu.VMEM((16,), jnp.int32),
                   pltpu.SMEM((16,), jnp.int32)])
def _sc_hist(eta_hbm, hist_hbm, eta_v, stage_v, h_s):
    w = _wid()
    pltpu.sync_copy(eta_hbm.at[pl.ds(w * _SLICE, _SLICE)], eta_v)
    for e in range(N_ETAS):
        h_s[e] = jnp.int32(0)
    for v in range(_NV):
        ev = eta_v[pl.ds(v * 16, 16)]
        for l in range(16):
            e = ev[l]
            h_s[e] = h_s[e] + 1
    lanes = lax.iota(jnp.int32, 16)
    hv = jnp.zeros((16,), jnp.int32)
    for e in range(N_ETAS):
        hv = jnp.where(lanes == e, h_s[e], hv)
    stage_v[...] = hv
    pltpu.sync_copy(stage_v, hist_hbm.at[pl.ds(w * 16, 16)])


@functools.partial(
    pl.kernel, mesh=_sc_mesh,
    out_type=[jax.ShapeDtypeStruct((N_COLS,), jnp.int32),
              jax.ShapeDtypeStruct((NT_PAD,), jnp.int32)],
    scratch_types=[pltpu.VMEM((_NW * 16,), jnp.int32),
                   pltpu.VMEM((_SLICE,), jnp.int32),
                   pltpu.VMEM((_SLICE,), jnp.int32),
                   pltpu.VMEM((NT_PAD,), jnp.int32),
                   pltpu.SMEM((32,), jnp.int32)])
def _sc_pos(eta_hbm, hist_hbm, pos_hbm, te_hbm,
            hist_v, eta_v, pos_v, te_v, sm):
    w = _wid()
    pltpu.sync_copy(hist_hbm, hist_v)
    pltpu.sync_copy(eta_hbm.at[pl.ds(w * _SLICE, _SLICE)], eta_v)
    tot_v = jnp.zeros((16,), jnp.int32)
    pre_s = [jnp.int32(0)] * N_ETAS
    for wp in range(_NW):
        hv = hist_v[pl.ds(wp * 16, 16)]
        tot_v = tot_v + hv
        before = jnp.int32(wp) < w
        for e in range(N_ETAS):
            pre_s[e] = pre_s[e] + jnp.where(before, hv[e], jnp.int32(0))
    run = jnp.int32(0)
    for e in range(N_ETAS):
        g = (tot_v[e] + (TILE_M - 1)) & (~(TILE_M - 1))
        sm[e] = run + pre_s[e]
        sm[8 + e] = run
        sm[16 + e] = g
        run = run + g

    lanes = lax.iota(jnp.int32, 16)
    for v in range(_NV):
        ev = eta_v[pl.ds(v * 16, 16)]
        pv = jnp.zeros((16,), jnp.int32)
        for l in range(16):
            e = ev[l]
            p = sm[e]
            sm[e] = p + 1
            pv = jnp.where(lanes == l, p, pv)
        pos_v[pl.ds(v * 16, 16)] = pv
    pltpu.sync_copy(pos_v, pos_hbm.at[pl.ds(w * _SLICE, _SLICE)])

    @pl.when(w == 0)
    def _te():
        for kv in range(NT_PAD // 16):
            tv = jnp.zeros((16,), jnp.int32)
            for l in range(16):
                s = (kv * 16 + l) * TILE_M
                acc = jnp.int32(0)
                for e in range(N_ETAS):
                    gb_e = sm[8 + e]
                    g_e = sm[16 + e]
                    inside = (gb_e <= s) & (s < gb_e + g_e)
                    acc = acc + jnp.where(inside, jnp.int32(e), jnp.int32(0))
                tv = jnp.where(lanes == l, acc, tv)
            te_v[pl.ds(kv * 16, 16)] = tv
        pltpu.sync_copy(te_v, te_hbm)


@functools.partial(
    pl.kernel, mesh=_sc_mesh,
    out_type=jax.ShapeDtypeStruct((P_PAD, C_IN), jnp.float32),
    scratch_types=[pltpu.VMEM((CHUNK,), jnp.int32),
                   pltpu.VMEM((CHUNK, C_IN), jnp.float32),
                   pltpu.SemaphoreType.DMA])
def _sc_scatter_rows(cols_hbm, pos_hbm, xs_hbm, idx_v, rows_v, sem):
    w = _wid()
    for ch in range(_SLICE // CHUNK):
        o = w * _SLICE + ch * CHUNK
        pltpu.sync_copy(pos_hbm.at[pl.ds(o, CHUNK)], idx_v)
        pltpu.sync_copy(cols_hbm.at[pl.ds(o, CHUNK)], rows_v)
        pltpu.async_copy(rows_v, xs_hbm.at[idx_v], sem).wait()


@functools.partial(
    pl.kernel, mesh=_sc_mesh,
    out_type=jax.ShapeDtypeStruct((N_COLS, D_OUT), jnp.float32),
    scratch_types=[pltpu.VMEM((CHUNK,), jnp.int32),
                   pltpu.VMEM((CHUNK, D_OUT), jnp.float32),
                   pltpu.SemaphoreType.DMA])
def _sc_gather_rows(outs_hbm, pos_hbm, oc_hbm, idx_v, rows_v, sem):
    w = _wid()
    for ch in range(_SLICE // CHUNK):
        o = w * _SLICE + ch * CHUNK
        pltpu.sync_copy(pos_hbm.at[pl.ds(o, CHUNK)], idx_v)
        pltpu.async_copy(outs_hbm.at[idx_v], rows_v, sem).wait()
        pltpu.sync_copy(rows_v, oc_hbm.at[pl.ds(o, CHUNK)])


H_BLK = 16
T_N = H_BLK * W_GRID


def _base_body(x_ref, bW_ref, bb_ref, bp_ref, cols_ref):
    xcat = jnp.concatenate(
        [x_ref[:, h, :] for h in range(H_BLK)], axis=1).astype(jnp.bfloat16)
    bp_ref[...] = jax.lax.dot_general(
        xcat, bW_ref[...].astype(jnp.bfloat16),
        (((0,), (0,)), ((), ())),
        preferred_element_type=jnp.float32) + bb_ref[...]
    r = lax.broadcasted_iota(jnp.int32, (C_IN, C_IN), 0)
    c = lax.broadcasted_iota(jnp.int32, (C_IN, C_IN), 1)
    eye = (r == c).astype(jnp.bfloat16)
    cols_ref[...] = jax.lax.dot_general(
        xcat, eye, (((0,), (0,)), ((), ())),
        preferred_element_type=jnp.float32)


def _moe_body(te_ref, xs_ref, eW_ref, eb_ref, os_ref):
    os_ref[...] = jax.lax.dot_general(
        xs_ref[...].astype(jnp.bfloat16), eW_ref[0].astype(jnp.bfloat16),
        (((1,), (0,)), ((), ())),
        preferred_element_type=jnp.float32) + eb_ref[0]


def kernel(x, eta, base_W, base_b, expert_W, expert_b):
    eta_flat = eta.reshape(N_COLS)
    bb2 = base_b.reshape(1, D_OUT)
    eb2 = expert_b.reshape(N_ETAS, 1, D_OUT)

    hist = _sc_hist(eta_flat)
    pos, te = _sc_pos(eta_flat, hist)

    bp, cols = pl.pallas_call(
        _base_body,
        grid=(H_GRID // H_BLK,),
        in_specs=[
            pl.BlockSpec((C_IN, H_BLK, W_GRID), lambda i: (0, i, 0)),
            pl.BlockSpec((C_IN, D_OUT), lambda i: (0, 0)),
            pl.BlockSpec((1, D_OUT), lambda i: (0, 0)),
        ],
        out_specs=[
            pl.BlockSpec((T_N, D_OUT), lambda i: (i, 0)),
            pl.BlockSpec((T_N, C_IN), lambda i: (i, 0)),
        ],
        out_shape=[
            jax.ShapeDtypeStruct((N_COLS, D_OUT), jnp.float32),
            jax.ShapeDtypeStruct((N_COLS, C_IN), jnp.float32),
        ],
        compiler_params=pltpu.CompilerParams(
            dimension_semantics=("parallel",)),
    )(x, base_W, bb2)

    xs = _sc_scatter_rows(cols, pos)

    out_s = pl.pallas_call(
        _moe_body,
        grid_spec=pltpu.PrefetchScalarGridSpec(
            num_scalar_prefetch=1,
            grid=(NT,),
            in_specs=[
                pl.BlockSpec((TILE_M, C_IN), lambda t, te_r: (t, 0)),
                pl.BlockSpec((1, C_IN, D_OUT), lambda t, te_r: (te_r[t], 0, 0)),
                pl.BlockSpec((1, 1, D_OUT), lambda t, te_r: (te_r[t], 0, 0)),
            ],
            out_specs=pl.BlockSpec((TILE_M, D_OUT), lambda t, te_r: (t, 0)),
        ),
        out_shape=jax.ShapeDtypeStruct((P_PAD, D_OUT), jnp.float32),
        compiler_params=pltpu.CompilerParams(
            dimension_semantics=("arbitrary",)),
    )(te, xs, expert_W, eb2)

    out_cols = _sc_gather_rows(out_s, pos)
    out3 = out_cols.T.reshape(D_OUT, H_GRID, W_GRID)
    return out3, bp


# speedup vs baseline: 1.0469x; 1.0469x over previous
import functools

import jax
import jax.numpy as jnp
from jax import lax
from jax.experimental import pallas as pl
from jax.experimental.pallas import tpu as pltpu
from jax.experimental.pallas import tpu_sc as plsc

C_IN, D_OUT, N_ETAS, H_GRID, W_GRID = 512, 512, 8, 64, 128
N_COLS = H_GRID * W_GRID

_NC, _NS = 2, 16
_NW = _NC * _NS
_SLICE = N_COLS // _NW
_NV = _SLICE // 16

TILE_M = 512
P_PAD = N_COLS + N_ETAS * TILE_M
NT = P_PAD // TILE_M
NT_PAD = 32

CHUNK = 128

_sc_mesh = plsc.VectorSubcoreMesh(core_axis_name="c", subcore_axis_name="s")


def _wid():
    return lax.axis_index("s") * _NC + lax.axis_index("c")


@functools.partial(
    pl.kernel, mesh=_sc_mesh,
    out_type=jax.ShapeDtypeStruct((_NW * 16,), jnp.int32),
    scratch_types=[pltpu.VMEM((_SLICE,), jnp.int32),
                   pltpu.VMEM((16,), jnp.int32),
                   pltpu.SMEM((16,), jnp.int32)])
def _sc_hist(eta_hbm, hist_hbm, eta_v, stage_v, h_s):
    w = _wid()
    pltpu.sync_copy(eta_hbm.at[pl.ds(w * _SLICE, _SLICE)], eta_v)
    for e in range(N_ETAS):
        h_s[e] = jnp.int32(0)
    for v in range(_NV):
        ev = eta_v[pl.ds(v * 16, 16)]
        for l in range(16):
            e = ev[l]
            h_s[e] = h_s[e] + 1
    lanes = lax.iota(jnp.int32, 16)
    hv = jnp.zeros((16,), jnp.int32)
    for e in range(N_ETAS):
        hv = jnp.where(lanes == e, h_s[e], hv)
    stage_v[...] = hv
    pltpu.sync_copy(stage_v, hist_hbm.at[pl.ds(w * 16, 16)])


@functools.partial(
    pl.kernel, mesh=_sc_mesh,
    out_type=[jax.ShapeDtypeStruct((N_COLS,), jnp.int32),
              jax.ShapeDtypeStruct((NT_PAD,), jnp.int32)],
    scratch_types=[pltpu.VMEM((_NW * 16,), jnp.int32),
                   pltpu.VMEM((_SLICE,), jnp.int32),
                   pltpu.VMEM((_SLICE,), jnp.int32),
                   pltpu.VMEM((NT_PAD,), jnp.int32),
                   pltpu.SMEM((32,), jnp.int32)])
def _sc_pos(eta_hbm, hist_hbm, pos_hbm, te_hbm,
            hist_v, eta_v, pos_v, te_v, sm):
    w = _wid()
    pltpu.sync_copy(hist_hbm, hist_v)
    pltpu.sync_copy(eta_hbm.at[pl.ds(w * _SLICE, _SLICE)], eta_v)
    tot_v = jnp.zeros((16,), jnp.int32)
    pre_s = [jnp.int32(0)] * N_ETAS
    for wp in range(_NW):
        hv = hist_v[pl.ds(wp * 16, 16)]
        tot_v = tot_v + hv
        before = jnp.int32(wp) < w
        for e in range(N_ETAS):
            pre_s[e] = pre_s[e] + jnp.where(before, hv[e], jnp.int32(0))
    run = jnp.int32(0)
    for e in range(N_ETAS):
        g = (tot_v[e] + (TILE_M - 1)) & (~(TILE_M - 1))
        sm[e] = run + pre_s[e]
        sm[8 + e] = run
        sm[16 + e] = g
        run = run + g

    lanes = lax.iota(jnp.int32, 16)
    for v in range(_NV):
        ev = eta_v[pl.ds(v * 16, 16)]
        pv = jnp.zeros((16,), jnp.int32)
        for l in range(16):
            e = ev[l]
            p = sm[e]
            sm[e] = p + 1
            pv = jnp.where(lanes == l, p, pv)
        pos_v[pl.ds(v * 16, 16)] = pv
    pltpu.sync_copy(pos_v, pos_hbm.at[pl.ds(w * _SLICE, _SLICE)])

    @pl.when(w == 0)
    def _te():
        for kv in range(NT_PAD // 16):
            tv = jnp.zeros((16,), jnp.int32)
            for l in range(16):
                s = (kv * 16 + l) * TILE_M
                acc = jnp.int32(0)
                for e in range(N_ETAS):
                    gb_e = sm[8 + e]
                    g_e = sm[16 + e]
                    inside = (gb_e <= s) & (s < gb_e + g_e)
                    acc = acc + jnp.where(inside, jnp.int32(e), jnp.int32(0))
                tv = jnp.where(lanes == l, acc, tv)
            te_v[pl.ds(kv * 16, 16)] = tv
        pltpu.sync_copy(te_v, te_hbm)


@functools.partial(
    pl.kernel, mesh=_sc_mesh,
    out_type=jax.ShapeDtypeStruct((P_PAD, C_IN), jnp.float32),
    scratch_types=[pltpu.VMEM((CHUNK,), jnp.int32),
                   pltpu.VMEM((CHUNK, C_IN), jnp.float32),
                   pltpu.SemaphoreType.DMA])
def _sc_scatter_rows(cols_hbm, pos_hbm, xs_hbm, idx_v, rows_v, sem):
    w = _wid()
    for ch in range(_SLICE // CHUNK):
        o = w * _SLICE + ch * CHUNK
        pltpu.sync_copy(pos_hbm.at[pl.ds(o, CHUNK)], idx_v)
        pltpu.sync_copy(cols_hbm.at[pl.ds(o, CHUNK)], rows_v)
        pltpu.async_copy(rows_v, xs_hbm.at[idx_v], sem).wait()


@functools.partial(
    pl.kernel, mesh=_sc_mesh,
    out_type=jax.ShapeDtypeStruct((N_COLS, D_OUT), jnp.float32),
    scratch_types=[pltpu.VMEM((CHUNK,), jnp.int32),
                   pltpu.VMEM((CHUNK, D_OUT), jnp.float32),
                   pltpu.SemaphoreType.DMA])
def _sc_gather_rows(outs_hbm, pos_hbm, oc_hbm, idx_v, rows_v, sem):
    w = _wid()
    for ch in range(_SLICE // CHUNK):
        o = w * _SLICE + ch * CHUNK
        pltpu.sync_copy(pos_hbm.at[pl.ds(o, CHUNK)], idx_v)
        pltpu.async_copy(outs_hbm.at[idx_v], rows_v, sem).wait()
        pltpu.sync_copy(rows_v, oc_hbm.at[pl.ds(o, CHUNK)])


H_BLK = 16
T_N = H_BLK * W_GRID


def _base_body(x_ref, bW_ref, bb_ref, bp_ref, cols_ref):
    xcat = jnp.concatenate(
        [x_ref[:, h, :] for h in range(H_BLK)], axis=1).astype(jnp.bfloat16)
    bp_ref[...] = jax.lax.dot_general(
        xcat, bW_ref[...].astype(jnp.bfloat16),
        (((0,), (0,)), ((), ())),
        preferred_element_type=jnp.float32) + bb_ref[...]
    r = lax.broadcasted_iota(jnp.int32, (C_IN, C_IN), 0)
    c = lax.broadcasted_iota(jnp.int32, (C_IN, C_IN), 1)
    eye = (r == c).astype(jnp.bfloat16)
    cols_ref[...] = jax.lax.dot_general(
        xcat, eye, (((0,), (0,)), ((), ())),
        preferred_element_type=jnp.float32)


def _moe_body(te_ref, xs_ref, eW_ref, eb_ref, os_ref):
    os_ref[...] = jax.lax.dot_general(
        xs_ref[...].astype(jnp.bfloat16), eW_ref[0].astype(jnp.bfloat16),
        (((1,), (0,)), ((), ())),
        preferred_element_type=jnp.float32) + eb_ref[0]


def kernel(x, eta, base_W, base_b, expert_W, expert_b):
    eta_flat = eta.reshape(N_COLS)
    bb2 = base_b.reshape(1, D_OUT)
    eb2 = expert_b.reshape(N_ETAS, 1, D_OUT)

    hist = _sc_hist(eta_flat)
    pos, te = _sc_pos(eta_flat, hist)

    bp, cols = pl.pallas_call(
        _base_body,
        grid=(H_GRID // H_BLK,),
        in_specs=[
            pl.BlockSpec((C_IN, H_BLK, W_GRID), lambda i: (0, i, 0)),
            pl.BlockSpec((C_IN, D_OUT), lambda i: (0, 0)),
            pl.BlockSpec((1, D_OUT), lambda i: (0, 0)),
        ],
        out_specs=[
            pl.BlockSpec((T_N, D_OUT), lambda i: (i, 0)),
            pl.BlockSpec((T_N, C_IN), lambda i: (i, 0)),
        ],
        out_shape=[
            jax.ShapeDtypeStruct((N_COLS, D_OUT), jnp.float32),
            jax.ShapeDtypeStruct((N_COLS, C_IN), jnp.float32),
        ],
        compiler_params=pltpu.CompilerParams(
            dimension_semantics=("parallel",)),
    )(x, base_W, bb2)

    xs = _sc_scatter_rows(cols, pos)

    out_s = pl.pallas_call(
        _moe_body,
        grid_spec=pltpu.PrefetchScalarGridSpec(
            num_scalar_prefetch=1,
            grid=(NT,),
            in_specs=[
                pl.BlockSpec((TILE_M, C_IN), lambda t, te_r: (t, 0)),
                pl.BlockSpec((1, C_IN, D_OUT), lambda t, te_r: (te_r[t], 0, 0)),
                pl.BlockSpec((1, 1, D_OUT), lambda t, te_r: (te_r[t], 0, 0)),
            ],
            out_specs=pl.BlockSpec((TILE_M, D_OUT), lambda t, te_r: (t, 0)),
        ),
        out_shape=jax.ShapeDtypeStruct((P_PAD, D_OUT), jnp.float32),
        compiler_params=pltpu.CompilerParams(
            dimension_semantics=("arbitrary",)),
    )(te, xs, expert_W, eb2)

    out_cols = _sc_gather_rows(out_s, pos)
    out3 = out_cols.T.reshape(D_OUT, H_GRID, W_GRID)
    return out3, bp
